# baseline (speedup 1.0000x reference)
import functools

import jax
import jax.numpy as jnp
from jax import lax
from jax.experimental import pallas as pl
from jax.experimental.pallas import tpu as pltpu
from jax.experimental.pallas import tpu_sc as plsc

_BATCH = 4
_SEQ = 8192
_DIM = 1024
_NW = 32
_ROWS_PER_W = _SEQ // _NW
_C = 32
_CHUNKS = _ROWS_PER_W // _C
_E = _C * _DIM
_STEPS = _CHUNKS * _BATCH

_mesh = plsc.VectorSubcoreMesh(core_axis_name="c", subcore_axis_name="s")


@functools.partial(
    pl.kernel,
    out_type=jax.ShapeDtypeStruct((_BATCH * _SEQ * _DIM,), jnp.float32),
    mesh=_mesh,
    scratch_types=[
        pltpu.VMEM((_E,), jnp.float32),
        pltpu.VMEM((_E,), jnp.float32),
        pltpu.VMEM((_E,), jnp.float32),
        pltpu.SemaphoreType.DMA,
        pltpu.SemaphoreType.DMA,
        pltpu.SemaphoreType.DMA,
        pltpu.SemaphoreType.DMA,
        pltpu.SemaphoreType.DMA,
    ],
)
def _sc_add(x_hbm, w_hbm, o_hbm, x0, x1, wv, si0, si1, so0, so1, sw):
    wid = lax.axis_index("s") * 2 + lax.axis_index("c")
    base_row = wid * _ROWS_PER_W
    xbufs = (x0, x1)
    isems = (si0, si1)
    osems = (so0, so1)

    def x_off(step):
        c, b = divmod(step, _BATCH)
        return b * (_SEQ * _DIM) + (base_row + c * _C) * _DIM

    in_copies = [None] * _STEPS
    out_copies = [None] * _STEPS

    w_copy = pltpu.async_copy(w_hbm.at[pl.ds(base_row * _DIM, _E)], wv, sw)
    in_copies[0] = pltpu.async_copy(
        x_hbm.at[pl.ds(x_off(0), _E)], xbufs[0], isems[0])

    for t in range(_STEPS):
        k = t % 2
        xb = xbufs[k]
        c, b = divmod(t, _BATCH)
        if t + 1 < _STEPS:
            if t >= 1:
                out_copies[t - 1].wait()
            kn = (t + 1) % 2
            in_copies[t + 1] = pltpu.async_copy(
                x_hbm.at[pl.ds(x_off(t + 1), _E)], xbufs[kn], isems[kn])
        in_copies[t].wait()
        if b == 0:
            w_copy.wait()

        def add_body(i, _):
            o = i * 64
            for j in range(4):
                sl = pl.ds(o + j * 16, 16)
                xb[sl] = xb[sl] + wv[sl]
            return 0

        lax.fori_loop(0, _E // 64, add_body, 0)

        if b == _BATCH - 1 and c + 1 < _CHUNKS:
            w_copy = pltpu.async_copy(
                w_hbm.at[pl.ds((base_row + (c + 1) * _C) * _DIM, _E)], wv, sw)
        out_copies[t] = pltpu.async_copy(
            xb, o_hbm.at[pl.ds(x_off(t), _E)], osems[k])

    out_copies[_STEPS - 2].wait()
    out_copies[_STEPS - 1].wait()


def kernel(x, weight):
    batch, seq_len, dim = x.shape
    xf = x.reshape(batch * seq_len * dim)
    wf = jax.lax.slice(weight, (0, 0), (seq_len, dim)).reshape(seq_len * dim)
    out = _sc_add(xf, wf)
    return out.reshape(batch, seq_len, dim)


# speedup vs baseline: 2.5419x; 2.5419x over previous
import functools

import jax
import jax.numpy as jnp
from jax import lax
from jax.experimental import pallas as pl
from jax.experimental.pallas import tpu as pltpu
from jax.experimental.pallas import tpu_sc as plsc

_BATCH = 4
_SEQ = 8192
_DIM = 1024
_NW = 32
_ROWS_PER_W = _SEQ // _NW
_C = 32
_CHUNKS = _ROWS_PER_W // _C
_STEPS = _CHUNKS * _BATCH

_mesh = plsc.VectorSubcoreMesh(core_axis_name="c", subcore_axis_name="s")


@functools.partial(
    pl.kernel,
    out_type=jax.ShapeDtypeStruct((_BATCH, _SEQ, _DIM), jnp.float32),
    mesh=_mesh,
    compiler_params=pltpu.CompilerParams(use_tc_tiling_on_sc=True),
    scratch_types=[
        pltpu.VMEM((_C, _DIM), jnp.float32),
        pltpu.VMEM((_C, _DIM), jnp.float32),
        pltpu.VMEM((_C, _DIM), jnp.float32),
        pltpu.SemaphoreType.DMA,
        pltpu.SemaphoreType.DMA,
        pltpu.SemaphoreType.DMA,
        pltpu.SemaphoreType.DMA,
        pltpu.SemaphoreType.DMA,
    ],
)
def _sc_add(x_hbm, w_hbm, o_hbm, x0, x1, wv, si0, si1, so0, so1, sw):
    wid = lax.axis_index("s") * 2 + lax.axis_index("c")
    base_row = wid * _ROWS_PER_W
    xbufs = (x0, x1)
    isems = (si0, si1)
    osems = (so0, so1)

    in_copies = [None] * _STEPS
    out_copies = [None] * _STEPS

    def x_slc(step):
        c, b = divmod(step, _BATCH)
        return (b, pl.ds(base_row + c * _C, _C), slice(None))

    w_copy = pltpu.async_copy(
        w_hbm.at[pl.ds(base_row, _C), :], wv, sw)
    in_copies[0] = pltpu.async_copy(x_hbm.at[x_slc(0)], xbufs[0], isems[0])

    for t in range(_STEPS):
        k = t % 2
        xb = xbufs[k]
        c, b = divmod(t, _BATCH)
        if t + 1 < _STEPS:
            if t >= 1:
                out_copies[t - 1].wait()
            kn = (t + 1) % 2
            in_copies[t + 1] = pltpu.async_copy(
                x_hbm.at[x_slc(t + 1)], xbufs[kn], isems[kn])
        in_copies[t].wait()
        if b == 0:
            w_copy.wait()

        def add_body(i, _):
            r = i // 16
            o = (i % 16) * 64
            for j in range(4):
                sl = pl.ds(o + j * 16, 16)
                xb[r, sl] = xb[r, sl] + wv[r, sl]
            return 0

        lax.fori_loop(0, _C * 16, add_body, 0)

        if b == _BATCH - 1 and c + 1 < _CHUNKS:
            w_copy = pltpu.async_copy(
                w_hbm.at[pl.ds(base_row + (c + 1) * _C, _C), :], wv, sw)
        out_copies[t] = pltpu.async_copy(xb, o_hbm.at[x_slc(t)], osems[k])

    out_copies[_STEPS - 2].wait()
    out_copies[_STEPS - 1].wait()


def kernel(x, weight):
    batch, seq_len, dim = x.shape
    wf = jax.lax.slice(weight, (0, 0), (seq_len, dim))
    return _sc_add(x, wf)
